# baseline (speedup 1.0000x reference)
import functools

import jax
import jax.numpy as jnp
from jax import lax
from jax.experimental import pallas as pl
from jax.experimental.pallas import tpu as pltpu
from jax.experimental.pallas import tpu_sc as plsc

N = 10000
E = 320000
D = 128
G = 64
H = 64
OUT = 10

NC = 2
NS = 16
NW = NC * NS
EPW = E // NW
CH = 128
NCH = 79
EPT = NCH * CH
NP = 10240
RPT = NP // NS
PAD_DST = N + 16

_MESH = dict(core_axis_name="c", subcore_axis_name="s", num_cores=NC,
             num_subcores=NS)

BR = 1000
NBLK = N // BR


@functools.cache
def _agg_sc_kernel(table_rows):
    return pl.kernel(
        _agg_body,
        out_type=jax.ShapeDtypeStruct((NC, NP, D), jnp.float32),
        mesh=plsc.VectorSubcoreMesh(**_MESH),
        scratch_types=[
            pltpu.VMEM_SHARED((NP, D), jnp.float32),
            pltpu.VMEM((NCH, CH), jnp.int32),
            pltpu.VMEM((1, CH), jnp.int32),
            pltpu.VMEM((1, CH), jnp.int32),
            pltpu.VMEM((CH, D), jnp.float32),
            pltpu.VMEM((CH, D), jnp.float32),
            pltpu.SemaphoreType.DMA,
            pltpu.SemaphoreType.DMA,
            pltpu.SemaphoreType.DMA,
            pltpu.SemaphoreType.DMA,
        ],
    )


def _agg_sc(table, src, dst):
    return _agg_sc_kernel(table.shape[0])(table, src, dst)


def _agg_body(tab_hbm, src_hbm, dst_hbm, out_hbm,
              agg_sp, sidx, didx0, didx1, rows0, rows1,
              sem0, sem1, sem2, sem3):
    c = lax.axis_index("c")
    s = lax.axis_index("s")
    wid = s * NC + c

    def fill_z(i, _):
        for k in range(D // 16):
            rows0[i, pl.ds(k * 16, 16)] = jnp.zeros((16,), jnp.float32)
        return 0
    lax.fori_loop(0, CH, fill_z, 0)
    for k in range(RPT // CH):
        pltpu.sync_copy(rows0, agg_sp.at[pl.ds(s * RPT + k * CH, CH)])

    pltpu.sync_copy(src_hbm.at[wid], sidx)
    plsc.subcore_barrier()

    pltpu.async_copy(tab_hbm.at[sidx.at[0]], rows0, sem0)
    pltpu.async_copy(dst_hbm.at[wid, pl.ds(0, 1)], didx0, sem2)

    def body(jj, _):
        j0 = 2 * jj
        j1 = j0 + 1
        pltpu.make_async_copy(tab_hbm.at[sidx.at[j0]], rows0, sem0).wait()
        pltpu.make_async_copy(dst_hbm.at[wid, pl.ds(j0, 1)], didx0,
                              sem2).wait()
        pltpu.async_copy(tab_hbm.at[sidx.at[j1]], rows1, sem1)
        pltpu.async_copy(dst_hbm.at[wid, pl.ds(j1, 1)], didx1, sem3)
        pltpu.sync_copy(rows0, agg_sp.at[didx0.at[0]], add=True)
        pltpu.make_async_copy(tab_hbm.at[sidx.at[j1]], rows1, sem1).wait()
        pltpu.make_async_copy(dst_hbm.at[wid, pl.ds(j1, 1)], didx1,
                              sem3).wait()
        pltpu.async_copy(tab_hbm.at[sidx.at[j0 + 2]], rows0, sem0)
        pltpu.async_copy(dst_hbm.at[wid, pl.ds(j0 + 2, 1)], didx0, sem2)
        pltpu.sync_copy(rows1, agg_sp.at[didx1.at[0]], add=True)
        return 0
    lax.fori_loop(0, (NCH - 1) // 2, body, 0)

    pltpu.make_async_copy(tab_hbm.at[sidx.at[NCH - 1]], rows0, sem0).wait()
    pltpu.make_async_copy(dst_hbm.at[wid, pl.ds(NCH - 1, 1)], didx0,
                          sem2).wait()
    pltpu.sync_copy(rows0, agg_sp.at[didx0.at[0]], add=True)

    plsc.subcore_barrier()
    for k in range(RPT // CH):
        pltpu.sync_copy(agg_sp.at[pl.ds(s * RPT + k * CH, CH)],
                        out_hbm.at[c, pl.ds(s * RPT + k * CH, CH)])


def _dinv_block(dega, degb):
    deg = dega[:, 0:1] + degb[:, 0:1] + 1.0
    return lax.rsqrt(deg)


def _tc0_body(x_ref, w_ref, dega_ref, degb_ref, hs_ref):
    dinv = _dinv_block(dega_ref, degb_ref)
    h = jnp.dot(x_ref[...], w_ref[...], preferred_element_type=jnp.float32,
                precision=lax.Precision.HIGHEST)
    hs_ref[...] = h * dinv


def _tc_first(x, w, dega, degb):
    return pl.pallas_call(
        _tc0_body,
        grid=(NBLK,),
        in_specs=[
            pl.BlockSpec((BR, D), lambda i: (i, 0)),
            pl.BlockSpec((D, D), lambda i: (0, 0)),
            pl.BlockSpec((BR, D), lambda i: (i, 0)),
            pl.BlockSpec((BR, D), lambda i: (i, 0)),
        ],
        out_specs=pl.BlockSpec((BR, D), lambda i: (i, 0)),
        out_shape=jax.ShapeDtypeStruct((N, D), jnp.float32),
    )(x, w, dega, degb)


def _tcmid_body(agga_ref, aggb_ref, hsp_ref, dega_ref, degb_ref, b_ref,
                w_ref, out_ref):
    dinv = _dinv_block(dega_ref, degb_ref)
    xl = dinv * (agga_ref[...] + aggb_ref[...] + hsp_ref[...]) + b_ref[...]
    xl = jnp.maximum(xl, 0.0)
    out_ref[...] = dinv * jnp.dot(xl, w_ref[...],
                                  preferred_element_type=jnp.float32,
                                  precision=lax.Precision.HIGHEST)


def _tc_mid(agga, aggb, hsp, dega, degb, b, w):
    return pl.pallas_call(
        _tcmid_body,
        grid=(NBLK,),
        in_specs=[
            pl.BlockSpec((BR, D), lambda i: (i, 0)),
            pl.BlockSpec((BR, D), lambda i: (i, 0)),
            pl.BlockSpec((BR, D), lambda i: (i, 0)),
            pl.BlockSpec((BR, D), lambda i: (i, 0)),
            pl.BlockSpec((BR, D), lambda i: (i, 0)),
            pl.BlockSpec((1, D), lambda i: (0, 0)),
            pl.BlockSpec((D, D), lambda i: (0, 0)),
        ],
        out_specs=pl.BlockSpec((BR, D), lambda i: (i, 0)),
        out_shape=jax.ShapeDtypeStruct((N, D), jnp.float32),
    )(agga, aggb, hsp, dega, degb, b, w)


def _tc3_body(agga_ref, aggb_ref, hsp_ref, dega_ref, degb_ref, b_ref,
              batch_ref, wm0_ref, bm0_ref, wm1_ref, bm1_ref,
              emb_ref, logits_ref, probs_ref, sums_s, cnt_s):
    i = pl.program_id(0)
    dinv = _dinv_block(dega_ref, degb_ref)
    emb = dinv * (agga_ref[...] + aggb_ref[...] + hsp_ref[...]) + b_ref[...]
    emb = jnp.maximum(emb, 0.0)
    emb_ref[...] = emb

    gids = lax.broadcasted_iota(jnp.int32, (1, G), 1)
    oh = (batch_ref[...] == gids).astype(jnp.float32)
    dn = (((0,), (0,)), ((), ()))
    psum = lax.dot_general(oh, emb, dn,
                           preferred_element_type=jnp.float32,
                           precision=lax.Precision.HIGHEST)
    ones = jnp.ones((BR, D), jnp.float32)
    pcnt = lax.dot_general(oh, ones, dn,
                           preferred_element_type=jnp.float32,
                           precision=lax.Precision.HIGHEST)

    @pl.when(i == 0)
    def _():
        sums_s[...] = jnp.zeros_like(sums_s)
        cnt_s[...] = jnp.zeros_like(cnt_s)

    sums_s[...] += psum
    cnt_s[...] += pcnt

    @pl.when(i == NBLK - 1)
    def _():
        pooled = sums_s[...] / jnp.maximum(cnt_s[...], 1.0)
        z = jnp.dot(pooled, wm0_ref[...],
                    preferred_element_type=jnp.float32,
                    precision=lax.Precision.HIGHEST) + bm0_ref[...]
        z = jnp.where(z > 0.0, z, jnp.exp(jnp.minimum(z, 0.0)) - 1.0)
        logits = jnp.dot(z, wm1_ref[...],
                         preferred_element_type=jnp.float32,
                         precision=lax.Precision.HIGHEST) + bm1_ref[...]
        logits_ref[...] = logits
        m = jnp.max(logits, axis=-1, keepdims=True)
        e = jnp.exp(logits - m)
        probs_ref[...] = e / jnp.sum(e, axis=-1, keepdims=True)


def _tc_last(agga, aggb, hsp, dega, degb, b, batch2, wm0, bm0, wm1, bm1):
    return pl.pallas_call(
        _tc3_body,
        grid=(NBLK,),
        in_specs=[
            pl.BlockSpec((BR, D), lambda i: (i, 0)),
            pl.BlockSpec((BR, D), lambda i: (i, 0)),
            pl.BlockSpec((BR, D), lambda i: (i, 0)),
            pl.BlockSpec((BR, D), lambda i: (i, 0)),
            pl.BlockSpec((BR, D), lambda i: (i, 0)),
            pl.BlockSpec((1, D), lambda i: (0, 0)),
            pl.BlockSpec((BR, 1), lambda i: (i, 0)),
            pl.BlockSpec((D, H), lambda i: (0, 0)),
            pl.BlockSpec((1, H), lambda i: (0, 0)),
            pl.BlockSpec((H, OUT), lambda i: (0, 0)),
            pl.BlockSpec((1, OUT), lambda i: (0, 0)),
        ],
        out_specs=[
            pl.BlockSpec((BR, D), lambda i: (i, 0)),
            pl.BlockSpec((G, OUT), lambda i: (0, 0)),
            pl.BlockSpec((G, OUT), lambda i: (0, 0)),
        ],
        out_shape=[
            jax.ShapeDtypeStruct((N, D), jnp.float32),
            jax.ShapeDtypeStruct((G, OUT), jnp.float32),
            jax.ShapeDtypeStruct((G, OUT), jnp.float32),
        ],
        scratch_shapes=[
            pltpu.VMEM((G, D), jnp.float32),
            pltpu.VMEM((G, D), jnp.float32),
        ],
    )(agga, aggb, hsp, dega, degb, b, batch2, wm0, bm0, wm1, bm1)


def kernel(x, edge_index, batch, W1, b1, W2, b2, W3, b3, Wm0, bm0, Wm1, bm1):
    pad = ((0, 0), (0, EPT - EPW))
    src = jnp.pad(edge_index[0].reshape(NW, EPW), pad).reshape(NW, NCH, CH)
    dst = jnp.pad(edge_index[1].reshape(NW, EPW), pad,
                  constant_values=PAD_DST).reshape(NW, NCH, CH)

    ones_tab = jnp.ones((8, D), jnp.float32)
    zsrc = jnp.zeros((NW, NCH, CH), jnp.int32)
    degp = _agg_sc(ones_tab, zsrc, dst)
    dega, degb = degp[0], degp[1]

    hs0 = _tc_first(x, W1, dega, degb)
    agg = _agg_sc(hs0, src, dst)
    hs1 = _tc_mid(agg[0], agg[1], hs0, dega, degb, b1.reshape(1, D), W2)
    agg = _agg_sc(hs1, src, dst)
    hs2 = _tc_mid(agg[0], agg[1], hs1, dega, degb, b2.reshape(1, D), W3)
    agg = _agg_sc(hs2, src, dst)
    emb, logits, probs = _tc_last(agg[0], agg[1], hs2, dega, degb,
                                  b3.reshape(1, D), batch.reshape(N, 1),
                                  Wm0, bm0.reshape(1, H), Wm1,
                                  bm1.reshape(1, OUT))
    return (logits, probs, emb)


# speedup vs baseline: 13.8807x; 13.8807x over previous
import functools

import jax
import jax.numpy as jnp
from jax import lax
from jax.experimental import pallas as pl
from jax.experimental.pallas import tpu as pltpu
from jax.experimental.pallas import tpu_sc as plsc

N = 10000
E = 320000
D = 128
G = 64
H = 64
OUT = 10

NC = 2
NS = 16
NW = NC * NS
EPW = E // NW
CH = 128
NCH = 79
EPT = NCH * CH
NP = 10240
RPT = NP // NS
PAD_DST = N + 16

_MESH = dict(core_axis_name="c", subcore_axis_name="s", num_cores=NC,
             num_subcores=NS)

BR = 1000
NBLK = N // BR


@functools.cache
def _agg_sc_kernel(table_rows):
    return pl.kernel(
        _agg_body,
        out_type=jax.ShapeDtypeStruct((NC, NP, D), jnp.float32),
        mesh=plsc.VectorSubcoreMesh(**_MESH),
        scratch_types=[
            pltpu.VMEM_SHARED((NP, D), jnp.float32),
            pltpu.VMEM((NCH, CH), jnp.int32),
            pltpu.VMEM((1, CH), jnp.int32),
            pltpu.VMEM((1, CH), jnp.int32),
            pltpu.VMEM((CH, D), jnp.float32),
            pltpu.VMEM((CH, D), jnp.float32),
            pltpu.SemaphoreType.DMA,
            pltpu.SemaphoreType.DMA,
            pltpu.SemaphoreType.DMA,
            pltpu.SemaphoreType.DMA,
        ],
    )


def _agg_sc(table, src, dst):
    return _agg_sc_kernel(table.shape[0])(table, src, dst)


@functools.cache
def _deg_sc_kernel():
    return pl.kernel(
        _deg_body,
        out_type=jax.ShapeDtypeStruct((NC, NP, 16), jnp.float32),
        mesh=plsc.VectorSubcoreMesh(**_MESH),
        scratch_types=[
            pltpu.VMEM_SHARED((NP, 16), jnp.float32),
            pltpu.VMEM((1, CH), jnp.int32),
            pltpu.VMEM((1, CH), jnp.int32),
            pltpu.VMEM((CH, 16), jnp.float32),
            pltpu.SemaphoreType.DMA,
            pltpu.SemaphoreType.DMA,
        ],
    )


def _deg_sc(dst):
    return _deg_sc_kernel()(dst)


def _deg_body(dst_hbm, out_hbm, hist_sp, didx0, didx1, ones_v, sem2, sem3):
    c = lax.axis_index("c")
    s = lax.axis_index("s")
    wid = s * NC + c

    def fill_z(i, _):
        ones_v[i, :] = jnp.zeros((16,), jnp.float32)
        return 0
    lax.fori_loop(0, CH, fill_z, 0)
    for k in range(RPT // CH):
        pltpu.sync_copy(ones_v, hist_sp.at[pl.ds(s * RPT + k * CH, CH)])

    def fill_o(i, _):
        ones_v[i, :] = jnp.ones((16,), jnp.float32)
        return 0
    lax.fori_loop(0, CH, fill_o, 0)
    plsc.subcore_barrier()

    pltpu.async_copy(dst_hbm.at[wid, pl.ds(0, 1)], didx0, sem2)

    def body(jj, _):
        j0 = 2 * jj
        j1 = j0 + 1
        pltpu.make_async_copy(dst_hbm.at[wid, pl.ds(j0, 1)], didx0,
                              sem2).wait()
        pltpu.async_copy(dst_hbm.at[wid, pl.ds(j1, 1)], didx1, sem3)
        pltpu.sync_copy(ones_v, hist_sp.at[didx0.at[0]], add=True)
        pltpu.make_async_copy(dst_hbm.at[wid, pl.ds(j1, 1)], didx1,
                              sem3).wait()
        pltpu.async_copy(dst_hbm.at[wid, pl.ds(j0 + 2, 1)], didx0, sem2)
        pltpu.sync_copy(ones_v, hist_sp.at[didx1.at[0]], add=True)
        return 0
    lax.fori_loop(0, (NCH - 1) // 2, body, 0)

    pltpu.make_async_copy(dst_hbm.at[wid, pl.ds(NCH - 1, 1)], didx0,
                          sem2).wait()
    pltpu.sync_copy(ones_v, hist_sp.at[didx0.at[0]], add=True)

    plsc.subcore_barrier()
    for k in range(RPT // CH):
        pltpu.sync_copy(hist_sp.at[pl.ds(s * RPT + k * CH, CH)],
                        out_hbm.at[c, pl.ds(s * RPT + k * CH, CH)])


def _agg_body(tab_hbm, src_hbm, dst_hbm, out_hbm,
              agg_sp, sidx, didx0, didx1, rows0, rows1,
              sem0, sem1, sem2, sem3):
    c = lax.axis_index("c")
    s = lax.axis_index("s")
    wid = s * NC + c

    def fill_z(i, _):
        for k in range(D // 16):
            rows0[i, pl.ds(k * 16, 16)] = jnp.zeros((16,), jnp.float32)
        return 0
    lax.fori_loop(0, CH, fill_z, 0)
    for k in range(RPT // CH):
        pltpu.sync_copy(rows0, agg_sp.at[pl.ds(s * RPT + k * CH, CH)])

    pltpu.sync_copy(src_hbm.at[wid], sidx)
    plsc.subcore_barrier()

    pltpu.async_copy(tab_hbm.at[sidx.at[0]], rows0, sem0)
    pltpu.async_copy(dst_hbm.at[wid, pl.ds(0, 1)], didx0, sem2)

    def body(jj, _):
        j0 = 2 * jj
        j1 = j0 + 1
        pltpu.make_async_copy(tab_hbm.at[sidx.at[j0]], rows0, sem0).wait()
        pltpu.make_async_copy(dst_hbm.at[wid, pl.ds(j0, 1)], didx0,
                              sem2).wait()
        pltpu.async_copy(tab_hbm.at[sidx.at[j1]], rows1, sem1)
        pltpu.async_copy(dst_hbm.at[wid, pl.ds(j1, 1)], didx1, sem3)
        pltpu.sync_copy(rows0, agg_sp.at[didx0.at[0]], add=True)
        pltpu.make_async_copy(tab_hbm.at[sidx.at[j1]], rows1, sem1).wait()
        pltpu.make_async_copy(dst_hbm.at[wid, pl.ds(j1, 1)], didx1,
                              sem3).wait()
        pltpu.async_copy(tab_hbm.at[sidx.at[j0 + 2]], rows0, sem0)
        pltpu.async_copy(dst_hbm.at[wid, pl.ds(j0 + 2, 1)], didx0, sem2)
        pltpu.sync_copy(rows1, agg_sp.at[didx1.at[0]], add=True)
        return 0
    lax.fori_loop(0, (NCH - 1) // 2, body, 0)

    pltpu.make_async_copy(tab_hbm.at[sidx.at[NCH - 1]], rows0, sem0).wait()
    pltpu.make_async_copy(dst_hbm.at[wid, pl.ds(NCH - 1, 1)], didx0,
                          sem2).wait()
    pltpu.sync_copy(rows0, agg_sp.at[didx0.at[0]], add=True)

    plsc.subcore_barrier()
    for k in range(RPT // CH):
        pltpu.sync_copy(agg_sp.at[pl.ds(s * RPT + k * CH, CH)],
                        out_hbm.at[c, pl.ds(s * RPT + k * CH, CH)])


def _dinv_block(dega, degb):
    deg = dega[:, 0:1] + degb[:, 0:1] + 1.0
    return lax.rsqrt(deg)


def _tc0_body(x_ref, w_ref, dega_ref, degb_ref, hs_ref):
    dinv = _dinv_block(dega_ref, degb_ref)
    h = jnp.dot(x_ref[...], w_ref[...], preferred_element_type=jnp.float32,
                precision=lax.Precision.HIGHEST)
    hs_ref[...] = h * dinv


def _tc_first(x, w, dega, degb):
    return pl.pallas_call(
        _tc0_body,
        grid=(NBLK,),
        in_specs=[
            pl.BlockSpec((BR, D), lambda i: (i, 0)),
            pl.BlockSpec((D, D), lambda i: (0, 0)),
            pl.BlockSpec((BR, 16), lambda i: (i, 0)),
            pl.BlockSpec((BR, 16), lambda i: (i, 0)),
        ],
        out_specs=pl.BlockSpec((BR, D), lambda i: (i, 0)),
        out_shape=jax.ShapeDtypeStruct((N, D), jnp.float32),
    )(x, w, dega, degb)


def _tcmid_body(agga_ref, aggb_ref, hsp_ref, dega_ref, degb_ref, b_ref,
                w_ref, out_ref):
    dinv = _dinv_block(dega_ref, degb_ref)
    xl = dinv * (agga_ref[...] + aggb_ref[...] + hsp_ref[...]) + b_ref[...]
    xl = jnp.maximum(xl, 0.0)
    out_ref[...] = dinv * jnp.dot(xl, w_ref[...],
                                  preferred_element_type=jnp.float32,
                                  precision=lax.Precision.HIGHEST)


def _tc_mid(agga, aggb, hsp, dega, degb, b, w):
    return pl.pallas_call(
        _tcmid_body,
        grid=(NBLK,),
        in_specs=[
            pl.BlockSpec((BR, D), lambda i: (i, 0)),
            pl.BlockSpec((BR, D), lambda i: (i, 0)),
            pl.BlockSpec((BR, D), lambda i: (i, 0)),
            pl.BlockSpec((BR, 16), lambda i: (i, 0)),
            pl.BlockSpec((BR, 16), lambda i: (i, 0)),
            pl.BlockSpec((1, D), lambda i: (0, 0)),
            pl.BlockSpec((D, D), lambda i: (0, 0)),
        ],
        out_specs=pl.BlockSpec((BR, D), lambda i: (i, 0)),
        out_shape=jax.ShapeDtypeStruct((N, D), jnp.float32),
    )(agga, aggb, hsp, dega, degb, b, w)


def _tc3_body(agga_ref, aggb_ref, hsp_ref, dega_ref, degb_ref, b_ref,
              batch_ref, wm0_ref, bm0_ref, wm1_ref, bm1_ref,
              emb_ref, logits_ref, probs_ref, sums_s, cnt_s):
    i = pl.program_id(0)
    dinv = _dinv_block(dega_ref, degb_ref)
    emb = dinv * (agga_ref[...] + aggb_ref[...] + hsp_ref[...]) + b_ref[...]
    emb = jnp.maximum(emb, 0.0)
    emb_ref[...] = emb

    gids = lax.broadcasted_iota(jnp.int32, (1, G), 1)
    oh = (batch_ref[...] == gids).astype(jnp.float32)
    dn = (((0,), (0,)), ((), ()))
    psum = lax.dot_general(oh, emb, dn,
                           preferred_element_type=jnp.float32,
                           precision=lax.Precision.HIGHEST)
    ones = jnp.ones((BR, D), jnp.float32)
    pcnt = lax.dot_general(oh, ones, dn,
                           preferred_element_type=jnp.float32,
                           precision=lax.Precision.HIGHEST)

    @pl.when(i == 0)
    def _():
        sums_s[...] = jnp.zeros_like(sums_s)
        cnt_s[...] = jnp.zeros_like(cnt_s)

    sums_s[...] += psum
    cnt_s[...] += pcnt

    @pl.when(i == NBLK - 1)
    def _():
        pooled = sums_s[...] / jnp.maximum(cnt_s[...], 1.0)
        z = jnp.dot(pooled, wm0_ref[...],
                    preferred_element_type=jnp.float32,
                    precision=lax.Precision.HIGHEST) + bm0_ref[...]
        z = jnp.where(z > 0.0, z, jnp.exp(jnp.minimum(z, 0.0)) - 1.0)
        logits = jnp.dot(z, wm1_ref[...],
                         preferred_element_type=jnp.float32,
                         precision=lax.Precision.HIGHEST) + bm1_ref[...]
        logits_ref[...] = logits
        m = jnp.max(logits, axis=-1, keepdims=True)
        e = jnp.exp(logits - m)
        probs_ref[...] = e / jnp.sum(e, axis=-1, keepdims=True)


def _tc_last(agga, aggb, hsp, dega, degb, b, batch2, wm0, bm0, wm1, bm1):
    return pl.pallas_call(
        _tc3_body,
        grid=(NBLK,),
        in_specs=[
            pl.BlockSpec((BR, D), lambda i: (i, 0)),
            pl.BlockSpec((BR, D), lambda i: (i, 0)),
            pl.BlockSpec((BR, D), lambda i: (i, 0)),
            pl.BlockSpec((BR, 16), lambda i: (i, 0)),
            pl.BlockSpec((BR, 16), lambda i: (i, 0)),
            pl.BlockSpec((1, D), lambda i: (0, 0)),
            pl.BlockSpec((BR, 1), lambda i: (i, 0)),
            pl.BlockSpec((D, H), lambda i: (0, 0)),
            pl.BlockSpec((1, H), lambda i: (0, 0)),
            pl.BlockSpec((H, OUT), lambda i: (0, 0)),
            pl.BlockSpec((1, OUT), lambda i: (0, 0)),
        ],
        out_specs=[
            pl.BlockSpec((BR, D), lambda i: (i, 0)),
            pl.BlockSpec((G, OUT), lambda i: (0, 0)),
            pl.BlockSpec((G, OUT), lambda i: (0, 0)),
        ],
        out_shape=[
            jax.ShapeDtypeStruct((N, D), jnp.float32),
            jax.ShapeDtypeStruct((G, OUT), jnp.float32),
            jax.ShapeDtypeStruct((G, OUT), jnp.float32),
        ],
        scratch_shapes=[
            pltpu.VMEM((G, D), jnp.float32),
            pltpu.VMEM((G, D), jnp.float32),
        ],
    )(agga, aggb, hsp, dega, degb, b, batch2, wm0, bm0, wm1, bm1)


def kernel(x, edge_index, batch, W1, b1, W2, b2, W3, b3, Wm0, bm0, Wm1, bm1):
    pad = ((0, 0), (0, EPT - EPW))
    src = jnp.pad(edge_index[0].reshape(NW, EPW), pad).reshape(NW, NCH, CH)
    dst = jnp.pad(edge_index[1].reshape(NW, EPW), pad,
                  constant_values=PAD_DST).reshape(NW, NCH, CH)

    degp = _deg_sc(dst)
    dega, degb = degp[0], degp[1]

    hs0 = _tc_first(x, W1, dega, degb)
    agg = _agg_sc(hs0, src, dst)
    hs1 = _tc_mid(agg[0], agg[1], hs0, dega, degb, b1.reshape(1, D), W2)
    agg = _agg_sc(hs1, src, dst)
    hs2 = _tc_mid(agg[0], agg[1], hs1, dega, degb, b2.reshape(1, D), W3)
    agg = _agg_sc(hs2, src, dst)
    emb, logits, probs = _tc_last(agg[0], agg[1], hs2, dega, degb,
                                  b3.reshape(1, D), batch.reshape(N, 1),
                                  Wm0, bm0.reshape(1, H), Wm1,
                                  bm1.reshape(1, OUT))
    return (logits, probs, emb)


# speedup vs baseline: 14.3688x; 1.0352x over previous
import functools

import jax
import jax.numpy as jnp
from jax import lax
from jax.experimental import pallas as pl
from jax.experimental.pallas import tpu as pltpu
from jax.experimental.pallas import tpu_sc as plsc

N = 10000
E = 320000
D = 128
G = 64
H = 64
OUT = 10

NC = 2
NS = 16
NW = NC * NS
EPW = E // NW
CH = 128
NCH = 79
EPT = NCH * CH
NP = 10240
RPT = NP // NS
PAD_DST = N + 16

_MESH = dict(core_axis_name="c", subcore_axis_name="s", num_cores=NC,
             num_subcores=NS)

BR = 1000
NBLK = N // BR


@functools.cache
def _agg_sc_kernel(table_rows):
    return pl.kernel(
        _agg_body,
        out_type=jax.ShapeDtypeStruct((NC, NP, D), jnp.float32),
        mesh=plsc.VectorSubcoreMesh(**_MESH),
        scratch_types=[
            pltpu.VMEM_SHARED((NP, D), jnp.float32),
            pltpu.VMEM((NCH, CH), jnp.int32),
            pltpu.VMEM((1, CH), jnp.int32),
            pltpu.VMEM((1, CH), jnp.int32),
            pltpu.VMEM((CH, D), jnp.float32),
            pltpu.VMEM((CH, D), jnp.float32),
            pltpu.SemaphoreType.DMA,
            pltpu.SemaphoreType.DMA,
            pltpu.SemaphoreType.DMA,
            pltpu.SemaphoreType.DMA,
        ],
    )


def _agg_sc(table, src, dst):
    return _agg_sc_kernel(table.shape[0])(table, src, dst)


@functools.cache
def _deg_sc_kernel():
    return pl.kernel(
        _deg_body,
        out_type=jax.ShapeDtypeStruct((NC, NP, 16), jnp.float32),
        mesh=plsc.VectorSubcoreMesh(**_MESH),
        scratch_types=[
            pltpu.VMEM_SHARED((NP, 16), jnp.float32),
            pltpu.VMEM((1, CH), jnp.int32),
            pltpu.VMEM((1, CH), jnp.int32),
            pltpu.VMEM((CH, 16), jnp.float32),
            pltpu.SemaphoreType.DMA,
            pltpu.SemaphoreType.DMA,
        ],
    )


def _deg_sc(dst):
    return _deg_sc_kernel()(dst)


def _deg_body(dst_hbm, out_hbm, hist_sp, didx0, didx1, ones_v, sem2, sem3):
    c = lax.axis_index("c")
    s = lax.axis_index("s")
    wid = s * NC + c

    def fill_z(i, _):
        ones_v[i, :] = jnp.zeros((16,), jnp.float32)
        return 0
    lax.fori_loop(0, CH, fill_z, 0)
    for k in range(RPT // CH):
        pltpu.sync_copy(ones_v, hist_sp.at[pl.ds(s * RPT + k * CH, CH)])

    def fill_o(i, _):
        ones_v[i, :] = jnp.ones((16,), jnp.float32)
        return 0
    lax.fori_loop(0, CH, fill_o, 0)
    plsc.subcore_barrier()

    pltpu.async_copy(dst_hbm.at[wid, pl.ds(0, 1)], didx0, sem2)

    def body(jj, _):
        j0 = 2 * jj
        j1 = j0 + 1
        pltpu.make_async_copy(dst_hbm.at[wid, pl.ds(j0, 1)], didx0,
                              sem2).wait()
        pltpu.async_copy(dst_hbm.at[wid, pl.ds(j1, 1)], didx1, sem3)
        pltpu.sync_copy(ones_v, hist_sp.at[didx0.at[0]], add=True)
        pltpu.make_async_copy(dst_hbm.at[wid, pl.ds(j1, 1)], didx1,
                              sem3).wait()
        pltpu.async_copy(dst_hbm.at[wid, pl.ds(j0 + 2, 1)], didx0, sem2)
        pltpu.sync_copy(ones_v, hist_sp.at[didx1.at[0]], add=True)
        return 0
    lax.fori_loop(0, (NCH - 1) // 2, body, 0)

    pltpu.make_async_copy(dst_hbm.at[wid, pl.ds(NCH - 1, 1)], didx0,
                          sem2).wait()
    pltpu.sync_copy(ones_v, hist_sp.at[didx0.at[0]], add=True)

    plsc.subcore_barrier()
    for k in range(RPT // CH):
        pltpu.sync_copy(hist_sp.at[pl.ds(s * RPT + k * CH, CH)],
                        out_hbm.at[c, pl.ds(s * RPT + k * CH, CH)])


def _agg_body(tab_hbm, src_hbm, dst_hbm, out_hbm,
              agg_sp, sidx, didx0, didx1, rows0, rows1,
              sem0, sem1, sem2, sem3):
    c = lax.axis_index("c")
    s = lax.axis_index("s")
    wid = s * NC + c

    def fill_z(i, _):
        for k in range(D // 16):
            rows0[i, pl.ds(k * 16, 16)] = jnp.zeros((16,), jnp.float32)
        return 0
    lax.fori_loop(0, CH, fill_z, 0)
    for k in range(RPT // CH):
        pltpu.sync_copy(rows0, agg_sp.at[pl.ds(s * RPT + k * CH, CH)])

    pltpu.sync_copy(src_hbm.at[wid], sidx)
    plsc.subcore_barrier()

    pltpu.async_copy(tab_hbm.at[sidx.at[0]], rows0, sem0)
    pltpu.async_copy(dst_hbm.at[wid, pl.ds(0, 1)], didx0, sem2)

    def body(jj, _):
        j0 = 2 * jj
        j1 = j0 + 1
        pltpu.make_async_copy(tab_hbm.at[sidx.at[j0]], rows0, sem0).wait()
        pltpu.make_async_copy(dst_hbm.at[wid, pl.ds(j0, 1)], didx0,
                              sem2).wait()
        pltpu.async_copy(tab_hbm.at[sidx.at[j1]], rows1, sem1)
        pltpu.async_copy(dst_hbm.at[wid, pl.ds(j1, 1)], didx1, sem3)
        pltpu.sync_copy(rows0, agg_sp.at[didx0.at[0]], add=True)
        pltpu.make_async_copy(tab_hbm.at[sidx.at[j1]], rows1, sem1).wait()
        pltpu.make_async_copy(dst_hbm.at[wid, pl.ds(j1, 1)], didx1,
                              sem3).wait()
        pltpu.async_copy(tab_hbm.at[sidx.at[j0 + 2]], rows0, sem0)
        pltpu.async_copy(dst_hbm.at[wid, pl.ds(j0 + 2, 1)], didx0, sem2)
        pltpu.sync_copy(rows1, agg_sp.at[didx1.at[0]], add=True)
        return 0
    lax.fori_loop(0, (NCH - 1) // 2, body, 0)

    pltpu.make_async_copy(tab_hbm.at[sidx.at[NCH - 1]], rows0, sem0).wait()
    pltpu.make_async_copy(dst_hbm.at[wid, pl.ds(NCH - 1, 1)], didx0,
                          sem2).wait()
    pltpu.sync_copy(rows0, agg_sp.at[didx0.at[0]], add=True)

    plsc.subcore_barrier()
    for k in range(RPT // CH):
        pltpu.sync_copy(agg_sp.at[pl.ds(s * RPT + k * CH, CH)],
                        out_hbm.at[c, pl.ds(s * RPT + k * CH, CH)])


def _dinv_block(dega, degb):
    deg = dega[:, 0:1] + degb[:, 0:1] + 1.0
    return lax.rsqrt(deg)


def _tc0_body(x_ref, w_ref, degp_ref, hs_ref):
    dinv = _dinv_block(degp_ref[0], degp_ref[1])
    h = jnp.dot(x_ref[...], w_ref[...], preferred_element_type=jnp.float32,
                precision=lax.Precision.HIGHEST)
    hs_ref[...] = h * dinv


def _tc_first(x, w, degp):
    return pl.pallas_call(
        _tc0_body,
        grid=(NBLK,),
        in_specs=[
            pl.BlockSpec((BR, D), lambda i: (i, 0)),
            pl.BlockSpec((D, D), lambda i: (0, 0)),
            pl.BlockSpec((NC, BR, 16), lambda i: (0, i, 0)),
        ],
        out_specs=pl.BlockSpec((BR, D), lambda i: (i, 0)),
        out_shape=jax.ShapeDtypeStruct((N, D), jnp.float32),
    )(x, w, degp)


def _tcmid_body(aggp_ref, hsp_ref, degp_ref, b_ref, w_ref, out_ref):
    dinv = _dinv_block(degp_ref[0], degp_ref[1])
    xl = dinv * (aggp_ref[0] + aggp_ref[1] + hsp_ref[...]) + b_ref[...]
    xl = jnp.maximum(xl, 0.0)
    out_ref[...] = dinv * jnp.dot(xl, w_ref[...],
                                  preferred_element_type=jnp.float32,
                                  precision=lax.Precision.HIGHEST)


def _tc_mid(aggp, hsp, degp, b, w):
    return pl.pallas_call(
        _tcmid_body,
        grid=(NBLK,),
        in_specs=[
            pl.BlockSpec((NC, BR, D), lambda i: (0, i, 0)),
            pl.BlockSpec((BR, D), lambda i: (i, 0)),
            pl.BlockSpec((NC, BR, 16), lambda i: (0, i, 0)),
            pl.BlockSpec((1, D), lambda i: (0, 0)),
            pl.BlockSpec((D, D), lambda i: (0, 0)),
        ],
        out_specs=pl.BlockSpec((BR, D), lambda i: (i, 0)),
        out_shape=jax.ShapeDtypeStruct((N, D), jnp.float32),
    )(aggp, hsp, degp, b, w)


def _tc3_body(aggp_ref, hsp_ref, degp_ref, b_ref,
              batch_ref, wm0_ref, bm0_ref, wm1_ref, bm1_ref,
              emb_ref, logits_ref, probs_ref, sums_s, cnt_s):
    i = pl.program_id(0)
    dinv = _dinv_block(degp_ref[0], degp_ref[1])
    emb = dinv * (aggp_ref[0] + aggp_ref[1] + hsp_ref[...]) + b_ref[...]
    emb = jnp.maximum(emb, 0.0)
    emb_ref[...] = emb

    gids = lax.broadcasted_iota(jnp.int32, (1, G), 1)
    oh = (batch_ref[...] == gids).astype(jnp.float32)
    dn = (((0,), (0,)), ((), ()))
    psum = lax.dot_general(oh, emb, dn,
                           preferred_element_type=jnp.float32,
                           precision=lax.Precision.HIGHEST)
    ones = jnp.ones((BR, D), jnp.float32)
    pcnt = lax.dot_general(oh, ones, dn,
                           preferred_element_type=jnp.float32,
                           precision=lax.Precision.HIGHEST)

    @pl.when(i == 0)
    def _():
        sums_s[...] = jnp.zeros_like(sums_s)
        cnt_s[...] = jnp.zeros_like(cnt_s)

    sums_s[...] += psum
    cnt_s[...] += pcnt

    @pl.when(i == NBLK - 1)
    def _():
        pooled = sums_s[...] / jnp.maximum(cnt_s[...], 1.0)
        z = jnp.dot(pooled, wm0_ref[...],
                    preferred_element_type=jnp.float32,
                    precision=lax.Precision.HIGHEST) + bm0_ref[...]
        z = jnp.where(z > 0.0, z, jnp.exp(jnp.minimum(z, 0.0)) - 1.0)
        logits = jnp.dot(z, wm1_ref[...],
                         preferred_element_type=jnp.float32,
                         precision=lax.Precision.HIGHEST) + bm1_ref[...]
        logits_ref[...] = logits
        m = jnp.max(logits, axis=-1, keepdims=True)
        e = jnp.exp(logits - m)
        probs_ref[...] = e / jnp.sum(e, axis=-1, keepdims=True)


def _tc_last(aggp, hsp, degp, b, batch2, wm0, bm0, wm1, bm1):
    return pl.pallas_call(
        _tc3_body,
        grid=(NBLK,),
        in_specs=[
            pl.BlockSpec((NC, BR, D), lambda i: (0, i, 0)),
            pl.BlockSpec((BR, D), lambda i: (i, 0)),
            pl.BlockSpec((NC, BR, 16), lambda i: (0, i, 0)),
            pl.BlockSpec((1, D), lambda i: (0, 0)),
            pl.BlockSpec((BR, 1), lambda i: (i, 0)),
            pl.BlockSpec((D, H), lambda i: (0, 0)),
            pl.BlockSpec((1, H), lambda i: (0, 0)),
            pl.BlockSpec((H, OUT), lambda i: (0, 0)),
            pl.BlockSpec((1, OUT), lambda i: (0, 0)),
        ],
        out_specs=[
            pl.BlockSpec((BR, D), lambda i: (i, 0)),
            pl.BlockSpec((G, OUT), lambda i: (0, 0)),
            pl.BlockSpec((G, OUT), lambda i: (0, 0)),
        ],
        out_shape=[
            jax.ShapeDtypeStruct((N, D), jnp.float32),
            jax.ShapeDtypeStruct((G, OUT), jnp.float32),
            jax.ShapeDtypeStruct((G, OUT), jnp.float32),
        ],
        scratch_shapes=[
            pltpu.VMEM((G, D), jnp.float32),
            pltpu.VMEM((G, D), jnp.float32),
        ],
    )(aggp, hsp, degp, b, batch2, wm0, bm0, wm1, bm1)


def kernel(x, edge_index, batch, W1, b1, W2, b2, W3, b3, Wm0, bm0, Wm1, bm1):
    pad = ((0, 0), (0, EPT - EPW))
    src = jnp.pad(edge_index[0].reshape(NW, EPW), pad).reshape(NW, NCH, CH)
    dst = jnp.pad(edge_index[1].reshape(NW, EPW), pad,
                  constant_values=PAD_DST).reshape(NW, NCH, CH)

    degp = _deg_sc(dst)

    hs0 = _tc_first(x, W1, degp)
    agg = _agg_sc(hs0, src, dst)
    hs1 = _tc_mid(agg, hs0, degp, b1.reshape(1, D), W2)
    agg = _agg_sc(hs1, src, dst)
    hs2 = _tc_mid(agg, hs1, degp, b2.reshape(1, D), W3)
    agg = _agg_sc(hs2, src, dst)
    emb, logits, probs = _tc_last(agg, hs2, degp,
                                  b3.reshape(1, D), batch.reshape(N, 1),
                                  Wm0, bm0.reshape(1, H), Wm1,
                                  bm1.reshape(1, OUT))
    return (logits, probs, emb)


# speedup vs baseline: 15.4424x; 1.0747x over previous
import functools

import jax
import jax.numpy as jnp
from jax import lax
from jax.experimental import pallas as pl
from jax.experimental.pallas import tpu as pltpu
from jax.experimental.pallas import tpu_sc as plsc

N = 10000
E = 320000
D = 128
G = 64
H = 64
OUT = 10

NC = 2
NS = 16
NW = NC * NS
EPW = E // NW
CH = 128
NCH = 79
EPT = NCH * CH
NP = 10240
RPT = NP // NS
PAD_DST = N + 16

_MESH = dict(core_axis_name="c", subcore_axis_name="s", num_cores=NC,
             num_subcores=NS)

BR = 1000
NBLK = N // BR


@functools.cache
def _agg_sc_kernel(table_rows):
    return pl.kernel(
        _agg_body,
        out_type=jax.ShapeDtypeStruct((NC, NP, D), jnp.float32),
        mesh=plsc.VectorSubcoreMesh(**_MESH),
        scratch_types=[
            pltpu.VMEM_SHARED((NP, D), jnp.float32),
            pltpu.VMEM((NCH, CH), jnp.int32),
            pltpu.VMEM((1, CH), jnp.int32),
            pltpu.VMEM((1, CH), jnp.int32),
            pltpu.VMEM((CH, D), jnp.float32),
            pltpu.VMEM((CH, D), jnp.float32),
            pltpu.SemaphoreType.DMA,
            pltpu.SemaphoreType.DMA,
            pltpu.SemaphoreType.DMA,
            pltpu.SemaphoreType.DMA,
        ],
    )


def _agg_sc(table, src, dst):
    return _agg_sc_kernel(table.shape[0])(table, src, dst)


@functools.cache
def _deg_sc_kernel():
    return pl.kernel(
        _deg_body,
        out_type=jax.ShapeDtypeStruct((NC, NP, 16), jnp.float32),
        mesh=plsc.VectorSubcoreMesh(**_MESH),
        scratch_types=[
            pltpu.VMEM_SHARED((NP, 16), jnp.float32),
            pltpu.VMEM((1, CH), jnp.int32),
            pltpu.VMEM((1, CH), jnp.int32),
            pltpu.VMEM((CH, 16), jnp.float32),
            pltpu.SemaphoreType.DMA,
            pltpu.SemaphoreType.DMA,
        ],
    )


def _deg_sc(dst):
    return _deg_sc_kernel()(dst)


def _deg_body(dst_hbm, out_hbm, hist_sp, didx0, didx1, ones_v, sem2, sem3):
    c = lax.axis_index("c")
    s = lax.axis_index("s")
    wid = s * NC + c

    def fill_z(i, _):
        ones_v[i, :] = jnp.zeros((16,), jnp.float32)
        return 0
    lax.fori_loop(0, CH, fill_z, 0)
    for k in range(RPT // CH):
        pltpu.sync_copy(ones_v, hist_sp.at[pl.ds(s * RPT + k * CH, CH)])

    def fill_o(i, _):
        ones_v[i, :] = jnp.ones((16,), jnp.float32)
        return 0
    lax.fori_loop(0, CH, fill_o, 0)
    plsc.subcore_barrier()

    pltpu.async_copy(dst_hbm.at[wid, pl.ds(0, 1)], didx0, sem2)

    def body(jj, _):
        j0 = 2 * jj
        j1 = j0 + 1
        pltpu.make_async_copy(dst_hbm.at[wid, pl.ds(j0, 1)], didx0,
                              sem2).wait()
        pltpu.async_copy(dst_hbm.at[wid, pl.ds(j1, 1)], didx1, sem3)
        pltpu.sync_copy(ones_v, hist_sp.at[didx0.at[0]], add=True)
        pltpu.make_async_copy(dst_hbm.at[wid, pl.ds(j1, 1)], didx1,
                              sem3).wait()
        pltpu.async_copy(dst_hbm.at[wid, pl.ds(j0 + 2, 1)], didx0, sem2)
        pltpu.sync_copy(ones_v, hist_sp.at[didx1.at[0]], add=True)
        return 0
    lax.fori_loop(0, (NCH - 1) // 2, body, 0)

    pltpu.make_async_copy(dst_hbm.at[wid, pl.ds(NCH - 1, 1)], didx0,
                          sem2).wait()
    pltpu.sync_copy(ones_v, hist_sp.at[didx0.at[0]], add=True)

    plsc.subcore_barrier()
    for k in range(RPT // CH):
        pltpu.sync_copy(hist_sp.at[pl.ds(s * RPT + k * CH, CH)],
                        out_hbm.at[c, pl.ds(s * RPT + k * CH, CH)])


def _agg_body(tab_hbm, src_hbm, dst_hbm, out_hbm,
              agg_sp, sidx, didx0, didx1, rows0, rows1,
              sem0, sem1, sem2, sem3):
    c = lax.axis_index("c")
    s = lax.axis_index("s")
    wid = s * NC + c

    def fill_z(i, _):
        for k in range(D // 16):
            rows0[i, pl.ds(k * 16, 16)] = jnp.zeros((16,), jnp.float32)
        return 0
    lax.fori_loop(0, CH, fill_z, 0)
    for k in range(RPT // CH):
        pltpu.sync_copy(rows0, agg_sp.at[pl.ds(s * RPT + k * CH, CH)])

    pltpu.sync_copy(src_hbm.at[wid], sidx)
    plsc.subcore_barrier()

    pltpu.async_copy(tab_hbm.at[sidx.at[0]], rows0, sem0)
    pltpu.async_copy(dst_hbm.at[wid, pl.ds(0, 1)], didx0, sem2)

    pltpu.async_copy(tab_hbm.at[sidx.at[1]], rows1, sem1)
    pltpu.async_copy(dst_hbm.at[wid, pl.ds(1, 1)], didx1, sem3)

    def body(jj, _):
        j0 = 2 * jj
        j1 = j0 + 1
        pltpu.make_async_copy(tab_hbm.at[sidx.at[j0]], rows0, sem0).wait()
        pltpu.make_async_copy(dst_hbm.at[wid, pl.ds(j0, 1)], didx0,
                              sem2).wait()
        pltpu.sync_copy(rows0, agg_sp.at[didx0.at[0]], add=True)
        pltpu.async_copy(dst_hbm.at[wid, pl.ds(j0 + 2, 1)], didx0, sem2)
        pltpu.async_copy(tab_hbm.at[sidx.at[j0 + 2]], rows0, sem0)
        pltpu.make_async_copy(tab_hbm.at[sidx.at[j1]], rows1, sem1).wait()
        pltpu.make_async_copy(dst_hbm.at[wid, pl.ds(j1, 1)], didx1,
                              sem3).wait()
        pltpu.sync_copy(rows1, agg_sp.at[didx1.at[0]], add=True)

        @pl.when(jj < (NCH - 1) // 2 - 1)
        def _():
            pltpu.async_copy(dst_hbm.at[wid, pl.ds(j1 + 2, 1)], didx1, sem3)
            pltpu.async_copy(tab_hbm.at[sidx.at[j1 + 2]], rows1, sem1)
        return 0
    lax.fori_loop(0, (NCH - 1) // 2, body, 0)

    pltpu.make_async_copy(tab_hbm.at[sidx.at[NCH - 1]], rows0, sem0).wait()
    pltpu.make_async_copy(dst_hbm.at[wid, pl.ds(NCH - 1, 1)], didx0,
                          sem2).wait()
    pltpu.sync_copy(rows0, agg_sp.at[didx0.at[0]], add=True)

    plsc.subcore_barrier()
    for k in range(RPT // CH):
        pltpu.sync_copy(agg_sp.at[pl.ds(s * RPT + k * CH, CH)],
                        out_hbm.at[c, pl.ds(s * RPT + k * CH, CH)])


def _dinv_block(dega, degb):
    deg = dega[:, 0:1] + degb[:, 0:1] + 1.0
    return lax.rsqrt(deg)


def _tc0_body(x_ref, w_ref, degp_ref, hs_ref):
    dinv = _dinv_block(degp_ref[0], degp_ref[1])
    h = jnp.dot(x_ref[...], w_ref[...], preferred_element_type=jnp.float32,
                precision=lax.Precision.HIGHEST)
    hs_ref[...] = h * dinv


def _tc_first(x, w, degp):
    return pl.pallas_call(
        _tc0_body,
        grid=(NBLK,),
        in_specs=[
            pl.BlockSpec((BR, D), lambda i: (i, 0)),
            pl.BlockSpec((D, D), lambda i: (0, 0)),
            pl.BlockSpec((NC, BR, 16), lambda i: (0, i, 0)),
        ],
        out_specs=pl.BlockSpec((BR, D), lambda i: (i, 0)),
        out_shape=jax.ShapeDtypeStruct((N, D), jnp.float32),
    )(x, w, degp)


def _tcmid_body(aggp_ref, hsp_ref, degp_ref, b_ref, w_ref, out_ref):
    dinv = _dinv_block(degp_ref[0], degp_ref[1])
    xl = dinv * (aggp_ref[0] + aggp_ref[1] + hsp_ref[...]) + b_ref[...]
    xl = jnp.maximum(xl, 0.0)
    out_ref[...] = dinv * jnp.dot(xl, w_ref[...],
                                  preferred_element_type=jnp.float32,
                                  precision=lax.Precision.HIGHEST)


def _tc_mid(aggp, hsp, degp, b, w):
    return pl.pallas_call(
        _tcmid_body,
        grid=(NBLK,),
        in_specs=[
            pl.BlockSpec((NC, BR, D), lambda i: (0, i, 0)),
            pl.BlockSpec((BR, D), lambda i: (i, 0)),
            pl.BlockSpec((NC, BR, 16), lambda i: (0, i, 0)),
            pl.BlockSpec((1, D), lambda i: (0, 0)),
            pl.BlockSpec((D, D), lambda i: (0, 0)),
        ],
        out_specs=pl.BlockSpec((BR, D), lambda i: (i, 0)),
        out_shape=jax.ShapeDtypeStruct((N, D), jnp.float32),
    )(aggp, hsp, degp, b, w)


def _tc3_body(aggp_ref, hsp_ref, degp_ref, b_ref,
              batch_ref, wm0_ref, bm0_ref, wm1_ref, bm1_ref,
              emb_ref, logits_ref, probs_ref, sums_s, cnt_s):
    i = pl.program_id(0)
    dinv = _dinv_block(degp_ref[0], degp_ref[1])
    emb = dinv * (aggp_ref[0] + aggp_ref[1] + hsp_ref[...]) + b_ref[...]
    emb = jnp.maximum(emb, 0.0)
    emb_ref[...] = emb

    gids = lax.broadcasted_iota(jnp.int32, (1, G), 1)
    oh = (batch_ref[...] == gids).astype(jnp.float32)
    dn = (((0,), (0,)), ((), ()))
    psum = lax.dot_general(oh, emb, dn,
                           preferred_element_type=jnp.float32,
                           precision=lax.Precision.HIGHEST)
    ones = jnp.ones((BR, D), jnp.float32)
    pcnt = lax.dot_general(oh, ones, dn,
                           preferred_element_type=jnp.float32,
                           precision=lax.Precision.HIGHEST)

    @pl.when(i == 0)
    def _():
        sums_s[...] = jnp.zeros_like(sums_s)
        cnt_s[...] = jnp.zeros_like(cnt_s)

    sums_s[...] += psum
    cnt_s[...] += pcnt

    @pl.when(i == NBLK - 1)
    def _():
        pooled = sums_s[...] / jnp.maximum(cnt_s[...], 1.0)
        z = jnp.dot(pooled, wm0_ref[...],
                    preferred_element_type=jnp.float32,
                    precision=lax.Precision.HIGHEST) + bm0_ref[...]
        z = jnp.where(z > 0.0, z, jnp.exp(jnp.minimum(z, 0.0)) - 1.0)
        logits = jnp.dot(z, wm1_ref[...],
                         preferred_element_type=jnp.float32,
                         precision=lax.Precision.HIGHEST) + bm1_ref[...]
        logits_ref[...] = logits
        m = jnp.max(logits, axis=-1, keepdims=True)
        e = jnp.exp(logits - m)
        probs_ref[...] = e / jnp.sum(e, axis=-1, keepdims=True)


def _tc_last(aggp, hsp, degp, b, batch2, wm0, bm0, wm1, bm1):
    return pl.pallas_call(
        _tc3_body,
        grid=(NBLK,),
        in_specs=[
            pl.BlockSpec((NC, BR, D), lambda i: (0, i, 0)),
            pl.BlockSpec((BR, D), lambda i: (i, 0)),
            pl.BlockSpec((NC, BR, 16), lambda i: (0, i, 0)),
            pl.BlockSpec((1, D), lambda i: (0, 0)),
            pl.BlockSpec((BR, 1), lambda i: (i, 0)),
            pl.BlockSpec((D, H), lambda i: (0, 0)),
            pl.BlockSpec((1, H), lambda i: (0, 0)),
            pl.BlockSpec((H, OUT), lambda i: (0, 0)),
            pl.BlockSpec((1, OUT), lambda i: (0, 0)),
        ],
        out_specs=[
            pl.BlockSpec((BR, D), lambda i: (i, 0)),
            pl.BlockSpec((G, OUT), lambda i: (0, 0)),
            pl.BlockSpec((G, OUT), lambda i: (0, 0)),
        ],
        out_shape=[
            jax.ShapeDtypeStruct((N, D), jnp.float32),
            jax.ShapeDtypeStruct((G, OUT), jnp.float32),
            jax.ShapeDtypeStruct((G, OUT), jnp.float32),
        ],
        scratch_shapes=[
            pltpu.VMEM((G, D), jnp.float32),
            pltpu.VMEM((G, D), jnp.float32),
        ],
    )(aggp, hsp, degp, b, batch2, wm0, bm0, wm1, bm1)


def kernel(x, edge_index, batch, W1, b1, W2, b2, W3, b3, Wm0, bm0, Wm1, bm1):
    pad = ((0, 0), (0, EPT - EPW))
    src = jnp.pad(edge_index[0].reshape(NW, EPW), pad).reshape(NW, NCH, CH)
    dst = jnp.pad(edge_index[1].reshape(NW, EPW), pad,
                  constant_values=PAD_DST).reshape(NW, NCH, CH)

    degp = _deg_sc(dst)

    hs0 = _tc_first(x, W1, degp)
    agg = _agg_sc(hs0, src, dst)
    hs1 = _tc_mid(agg, hs0, degp, b1.reshape(1, D), W2)
    agg = _agg_sc(hs1, src, dst)
    hs2 = _tc_mid(agg, hs1, degp, b2.reshape(1, D), W3)
    agg = _agg_sc(hs2, src, dst)
    emb, logits, probs = _tc_last(agg, hs2, degp,
                                  b3.reshape(1, D), batch.reshape(N, 1),
                                  Wm0, bm0.reshape(1, H), Wm1,
                                  bm1.reshape(1, OUT))
    return (logits, probs, emb)


# speedup vs baseline: 15.5576x; 1.0075x over previous
import functools

import jax
import jax.numpy as jnp
from jax import lax
from jax.experimental import pallas as pl
from jax.experimental.pallas import tpu as pltpu
from jax.experimental.pallas import tpu_sc as plsc

N = 10000
E = 320000
D = 128
G = 64
H = 64
OUT = 10

NC = 2
NS = 16
NW = NC * NS
EPW = E // NW
CH = 128
NCH = 79
EPT = NCH * CH
NP = 10240
RPT = NP // NS
PAD_DST = N + 16

_MESH = dict(core_axis_name="c", subcore_axis_name="s", num_cores=NC,
             num_subcores=NS)

BR = 1000
NBLK = N // BR


@functools.cache
def _agg_sc_kernel(table_rows):
    return pl.kernel(
        _agg_body,
        out_type=jax.ShapeDtypeStruct((NC, NP, D), jnp.float32),
        mesh=plsc.VectorSubcoreMesh(**_MESH),
        scratch_types=[
            pltpu.VMEM_SHARED((NP, D), jnp.float32),
            pltpu.VMEM((NCH, CH), jnp.int32),
            pltpu.VMEM((1, CH), jnp.int32),
            pltpu.VMEM((1, CH), jnp.int32),
            pltpu.VMEM((CH, D), jnp.float32),
            pltpu.VMEM((CH, D), jnp.float32),
            pltpu.VMEM((32, D), jnp.float32),
            pltpu.SemaphoreType.DMA,
            pltpu.SemaphoreType.DMA,
            pltpu.SemaphoreType.DMA,
            pltpu.SemaphoreType.DMA,
        ],
    )


def _agg_sc(table, src, dst):
    return _agg_sc_kernel(table.shape[0])(table, src, dst)


@functools.cache
def _deg_sc_kernel():
    return pl.kernel(
        _deg_body,
        out_type=jax.ShapeDtypeStruct((NC, NP, 16), jnp.float32),
        mesh=plsc.VectorSubcoreMesh(**_MESH),
        scratch_types=[
            pltpu.VMEM_SHARED((NP, 16), jnp.float32),
            pltpu.VMEM((1, CH), jnp.int32),
            pltpu.VMEM((1, CH), jnp.int32),
            pltpu.VMEM((CH, 16), jnp.float32),
            pltpu.SemaphoreType.DMA,
            pltpu.SemaphoreType.DMA,
        ],
    )


def _deg_sc(dst):
    return _deg_sc_kernel()(dst)


def _deg_body(dst_hbm, out_hbm, hist_sp, didx0, didx1, ones_v, sem2, sem3):
    c = lax.axis_index("c")
    s = lax.axis_index("s")
    wid = s * NC + c

    def fill_z(i, _):
        ones_v[i, :] = jnp.zeros((16,), jnp.float32)
        return 0
    lax.fori_loop(0, CH, fill_z, 0)
    for k in range(RPT // CH):
        pltpu.sync_copy(ones_v, hist_sp.at[pl.ds(s * RPT + k * CH, CH)])

    def fill_o(i, _):
        ones_v[i, :] = jnp.ones((16,), jnp.float32)
        return 0
    lax.fori_loop(0, CH, fill_o, 0)
    plsc.subcore_barrier()

    pltpu.async_copy(dst_hbm.at[wid, pl.ds(0, 1)], didx0, sem2)

    def body(jj, _):
        j0 = 2 * jj
        j1 = j0 + 1
        pltpu.make_async_copy(dst_hbm.at[wid, pl.ds(j0, 1)], didx0,
                              sem2).wait()
        pltpu.async_copy(dst_hbm.at[wid, pl.ds(j1, 1)], didx1, sem3)
        pltpu.sync_copy(ones_v, hist_sp.at[didx0.at[0]], add=True)
        pltpu.make_async_copy(dst_hbm.at[wid, pl.ds(j1, 1)], didx1,
                              sem3).wait()
        pltpu.async_copy(dst_hbm.at[wid, pl.ds(j0 + 2, 1)], didx0, sem2)
        pltpu.sync_copy(ones_v, hist_sp.at[didx1.at[0]], add=True)
        return 0
    lax.fori_loop(0, (NCH - 1) // 2, body, 0)

    pltpu.make_async_copy(dst_hbm.at[wid, pl.ds(NCH - 1, 1)], didx0,
                          sem2).wait()
    pltpu.sync_copy(ones_v, hist_sp.at[didx0.at[0]], add=True)

    plsc.subcore_barrier()
    for k in range(RPT // CH):
        pltpu.sync_copy(hist_sp.at[pl.ds(s * RPT + k * CH, CH)],
                        out_hbm.at[c, pl.ds(s * RPT + k * CH, CH)])


def _agg_body(tab_hbm, src_hbm, dst_hbm, out_hbm,
              agg_sp, sidx, didx0, didx1, rows0, rows1, zb,
              sem0, sem1, sem2, sem3):
    c = lax.axis_index("c")
    s = lax.axis_index("s")
    wid = s * NC + c

    pltpu.sync_copy(src_hbm.at[wid], sidx)
    pltpu.async_copy(tab_hbm.at[sidx.at[0]], rows0, sem0)
    pltpu.async_copy(dst_hbm.at[wid, pl.ds(0, 1)], didx0, sem2)
    pltpu.async_copy(tab_hbm.at[sidx.at[1]], rows1, sem1)
    pltpu.async_copy(dst_hbm.at[wid, pl.ds(1, 1)], didx1, sem3)

    def fill_z(i, _):
        for k in range(D // 16):
            zb[i, pl.ds(k * 16, 16)] = jnp.zeros((16,), jnp.float32)
        return 0
    lax.fori_loop(0, 32, fill_z, 0)
    for k in range(RPT // 32):
        pltpu.sync_copy(zb, agg_sp.at[pl.ds(s * RPT + k * 32, 32)])
    plsc.subcore_barrier()

    def body(jj, _):
        j0 = 2 * jj
        j1 = j0 + 1
        pltpu.make_async_copy(tab_hbm.at[sidx.at[j0]], rows0, sem0).wait()
        pltpu.make_async_copy(dst_hbm.at[wid, pl.ds(j0, 1)], didx0,
                              sem2).wait()
        pltpu.sync_copy(rows0, agg_sp.at[didx0.at[0]], add=True)
        pltpu.async_copy(dst_hbm.at[wid, pl.ds(j0 + 2, 1)], didx0, sem2)
        pltpu.async_copy(tab_hbm.at[sidx.at[j0 + 2]], rows0, sem0)
        pltpu.make_async_copy(tab_hbm.at[sidx.at[j1]], rows1, sem1).wait()
        pltpu.make_async_copy(dst_hbm.at[wid, pl.ds(j1, 1)], didx1,
                              sem3).wait()
        pltpu.sync_copy(rows1, agg_sp.at[didx1.at[0]], add=True)

        @pl.when(jj < (NCH - 1) // 2 - 1)
        def _():
            pltpu.async_copy(dst_hbm.at[wid, pl.ds(j1 + 2, 1)], didx1, sem3)
            pltpu.async_copy(tab_hbm.at[sidx.at[j1 + 2]], rows1, sem1)
        return 0
    lax.fori_loop(0, (NCH - 1) // 2, body, 0)

    pltpu.make_async_copy(tab_hbm.at[sidx.at[NCH - 1]], rows0, sem0).wait()
    pltpu.make_async_copy(dst_hbm.at[wid, pl.ds(NCH - 1, 1)], didx0,
                          sem2).wait()
    pltpu.sync_copy(rows0, agg_sp.at[didx0.at[0]], add=True)

    plsc.subcore_barrier()
    for k in range(RPT // CH):
        pltpu.sync_copy(agg_sp.at[pl.ds(s * RPT + k * CH, CH)],
                        out_hbm.at[c, pl.ds(s * RPT + k * CH, CH)])


def _dinv_block(dega, degb):
    deg = dega[:, 0:1] + degb[:, 0:1] + 1.0
    return lax.rsqrt(deg)


def _tc0a_body(x_ref, w_ref, h_ref):
    h_ref[...] = jnp.dot(x_ref[...], w_ref[...],
                         preferred_element_type=jnp.float32,
                         precision=lax.Precision.HIGHEST)


def _tc0b_body(h_ref, degp_ref, hs_ref):
    dinv = _dinv_block(degp_ref[0], degp_ref[1])
    hs_ref[...] = h_ref[...] * dinv


def _tc_first(x, w, degp):
    h = pl.pallas_call(
        _tc0a_body,
        grid=(NBLK,),
        in_specs=[
            pl.BlockSpec((BR, D), lambda i: (i, 0)),
            pl.BlockSpec((D, D), lambda i: (0, 0)),
        ],
        out_specs=pl.BlockSpec((BR, D), lambda i: (i, 0)),
        out_shape=jax.ShapeDtypeStruct((N, D), jnp.float32),
    )(x, w)
    return pl.pallas_call(
        _tc0b_body,
        grid=(NBLK,),
        in_specs=[
            pl.BlockSpec((BR, D), lambda i: (i, 0)),
            pl.BlockSpec((NC, BR, 16), lambda i: (0, i, 0)),
        ],
        out_specs=pl.BlockSpec((BR, D), lambda i: (i, 0)),
        out_shape=jax.ShapeDtypeStruct((N, D), jnp.float32),
    )(h, degp)


def _tcmid_body(aggp_ref, hsp_ref, degp_ref, b_ref, w_ref, out_ref):
    dinv = _dinv_block(degp_ref[0], degp_ref[1])
    xl = dinv * (aggp_ref[0] + aggp_ref[1] + hsp_ref[...]) + b_ref[...]
    xl = jnp.maximum(xl, 0.0)
    out_ref[...] = dinv * jnp.dot(xl, w_ref[...],
                                  preferred_element_type=jnp.float32,
                                  precision=lax.Precision.HIGHEST)


def _tc_mid(aggp, hsp, degp, b, w):
    return pl.pallas_call(
        _tcmid_body,
        grid=(NBLK,),
        in_specs=[
            pl.BlockSpec((NC, BR, D), lambda i: (0, i, 0)),
            pl.BlockSpec((BR, D), lambda i: (i, 0)),
            pl.BlockSpec((NC, BR, 16), lambda i: (0, i, 0)),
            pl.BlockSpec((1, D), lambda i: (0, 0)),
            pl.BlockSpec((D, D), lambda i: (0, 0)),
        ],
        out_specs=pl.BlockSpec((BR, D), lambda i: (i, 0)),
        out_shape=jax.ShapeDtypeStruct((N, D), jnp.float32),
    )(aggp, hsp, degp, b, w)


def _tc3_body(aggp_ref, hsp_ref, degp_ref, b_ref,
              batch_ref, wm0_ref, bm0_ref, wm1_ref, bm1_ref,
              emb_ref, logits_ref, probs_ref, sums_s, cnt_s):
    i = pl.program_id(0)
    dinv = _dinv_block(degp_ref[0], degp_ref[1])
    emb = dinv * (aggp_ref[0] + aggp_ref[1] + hsp_ref[...]) + b_ref[...]
    emb = jnp.maximum(emb, 0.0)
    emb_ref[...] = emb

    gids = lax.broadcasted_iota(jnp.int32, (1, G), 1)
    oh = (batch_ref[...] == gids).astype(jnp.float32)
    dn = (((0,), (0,)), ((), ()))
    psum = lax.dot_general(oh, emb, dn,
                           preferred_element_type=jnp.float32,
                           precision=lax.Precision.HIGHEST)
    ones = jnp.ones((BR, D), jnp.float32)
    pcnt = lax.dot_general(oh, ones, dn,
                           preferred_element_type=jnp.float32,
                           precision=lax.Precision.HIGHEST)

    @pl.when(i == 0)
    def _():
        sums_s[...] = jnp.zeros_like(sums_s)
        cnt_s[...] = jnp.zeros_like(cnt_s)

    sums_s[...] += psum
    cnt_s[...] += pcnt

    @pl.when(i == NBLK - 1)
    def _():
        pooled = sums_s[...] / jnp.maximum(cnt_s[...], 1.0)
        z = jnp.dot(pooled, wm0_ref[...],
                    preferred_element_type=jnp.float32,
                    precision=lax.Precision.HIGHEST) + bm0_ref[...]
        z = jnp.where(z > 0.0, z, jnp.exp(jnp.minimum(z, 0.0)) - 1.0)
        logits = jnp.dot(z, wm1_ref[...],
                         preferred_element_type=jnp.float32,
                         precision=lax.Precision.HIGHEST) + bm1_ref[...]
        logits_ref[...] = logits
        m = jnp.max(logits, axis=-1, keepdims=True)
        e = jnp.exp(logits - m)
        probs_ref[...] = e / jnp.sum(e, axis=-1, keepdims=True)


def _tc_last(aggp, hsp, degp, b, batch2, wm0, bm0, wm1, bm1):
    return pl.pallas_call(
        _tc3_body,
        grid=(NBLK,),
        in_specs=[
            pl.BlockSpec((NC, BR, D), lambda i: (0, i, 0)),
            pl.BlockSpec((BR, D), lambda i: (i, 0)),
            pl.BlockSpec((NC, BR, 16), lambda i: (0, i, 0)),
            pl.BlockSpec((1, D), lambda i: (0, 0)),
            pl.BlockSpec((BR, 1), lambda i: (i, 0)),
            pl.BlockSpec((D, H), lambda i: (0, 0)),
            pl.BlockSpec((1, H), lambda i: (0, 0)),
            pl.BlockSpec((H, OUT), lambda i: (0, 0)),
            pl.BlockSpec((1, OUT), lambda i: (0, 0)),
        ],
        out_specs=[
            pl.BlockSpec((BR, D), lambda i: (i, 0)),
            pl.BlockSpec((G, OUT), lambda i: (0, 0)),
            pl.BlockSpec((G, OUT), lambda i: (0, 0)),
        ],
        out_shape=[
            jax.ShapeDtypeStruct((N, D), jnp.float32),
            jax.ShapeDtypeStruct((G, OUT), jnp.float32),
            jax.ShapeDtypeStruct((G, OUT), jnp.float32),
        ],
        scratch_shapes=[
            pltpu.VMEM((G, D), jnp.float32),
            pltpu.VMEM((G, D), jnp.float32),
        ],
    )(aggp, hsp, degp, b, batch2, wm0, bm0, wm1, bm1)


def kernel(x, edge_index, batch, W1, b1, W2, b2, W3, b3, Wm0, bm0, Wm1, bm1):
    pad = ((0, 0), (0, EPT - EPW))
    src = jnp.pad(edge_index[0].reshape(NW, EPW), pad).reshape(NW, NCH, CH)
    dst = jnp.pad(edge_index[1].reshape(NW, EPW), pad,
                  constant_values=PAD_DST).reshape(NW, NCH, CH)

    degp = _deg_sc(dst)

    hs0 = _tc_first(x, W1, degp)
    agg = _agg_sc(hs0, src, dst)
    hs1 = _tc_mid(agg, hs0, degp, b1.reshape(1, D), W2)
    agg = _agg_sc(hs1, src, dst)
    hs2 = _tc_mid(agg, hs1, degp, b2.reshape(1, D), W3)
    agg = _agg_sc(hs2, src, dst)
    emb, logits, probs = _tc_last(agg, hs2, degp,
                                  b3.reshape(1, D), batch.reshape(N, 1),
                                  Wm0, bm0.reshape(1, H), Wm1,
                                  bm1.reshape(1, OUT))
    return (logits, probs, emb)
